# baseline (speedup 1.0000x reference)
import functools

import jax
import jax.numpy as jnp
from jax import lax
from jax.experimental import pallas as pl
from jax.experimental.pallas import tpu as pltpu
from jax.experimental.pallas import tpu_sc as plsc

B = 16384
EMBED = 64
FEAT = 256
NUM_USERS = 1000000

_info = plsc.get_sparse_core_info()
_NC = _info.num_cores
_NS = _info.num_subcores
_NW = _NC * _NS
_BPW = B // _NW

_ICH = 64
_NIC = _BPW // _ICH
_UG = 64
_NUG = _BPW // _UG

_mesh = plsc.VectorSubcoreMesh(core_axis_name="c", subcore_axis_name="s")


@functools.partial(
    pl.kernel,
    mesh=_mesh,
    compiler_params=pltpu.CompilerParams(needs_layout_passes=False,
                                         skip_device_barrier=True),
    out_type=jax.ShapeDtypeStruct((B, FEAT), jnp.float32),
    scratch_types=[
        pltpu.VMEM((_BPW,), jnp.int32),
        pltpu.VMEM((_ICH, FEAT), jnp.float32),
        pltpu.VMEM((_ICH, FEAT), jnp.float32),
        pltpu.SemaphoreType.DMA,
        pltpu.SemaphoreType.DMA,
        pltpu.SemaphoreType.DMA,
        pltpu.SemaphoreType.DMA,
    ],
)
def _sc_item_gather(item_ids_hbm, item_feats_hbm, item_rows_hbm,
                    iids_v, ibuf0, ibuf1, igs0, igs1, iws0, iws1):
    wid = lax.axis_index("s") * _NC + lax.axis_index("c")
    base = wid * _BPW
    pltpu.sync_copy(item_ids_hbm.at[pl.ds(base, _BPW)], iids_v)

    ibufs = (ibuf0, ibuf1)
    igs = (igs0, igs1)
    iws = (iws0, iws1)

    def _gather(g, buf, sem):
        return pltpu.async_copy(
            item_feats_hbm.at[iids_v.at[pl.ds(g * _ICH, _ICH)]], buf, sem)

    igh = [None, None]
    iwh = [None, None]
    igh[0] = _gather(0, ibufs[0], igs[0])
    for g in range(_NIC):
        b = g % 2
        if g + 1 < _NIC:
            if iwh[1 - b] is not None:
                iwh[1 - b].wait()
            igh[1 - b] = _gather(g + 1, ibufs[1 - b], igs[1 - b])
        igh[b].wait()
        iwh[b] = pltpu.async_copy(
            ibufs[b], item_rows_hbm.at[pl.ds(base + g * _ICH, _ICH)], iws[b])
    iwh[0].wait()
    iwh[1].wait()


@functools.partial(
    pl.kernel,
    mesh=_mesh,
    compiler_params=pltpu.CompilerParams(needs_layout_passes=False,
                                         skip_device_barrier=True),
    out_type=jax.ShapeDtypeStruct((B, EMBED), jnp.float32),
    scratch_types=[
        pltpu.VMEM((_BPW,), jnp.int32),
        pltpu.VMEM((_UG, EMBED), jnp.float32),
        pltpu.VMEM((_UG, EMBED), jnp.float32),
        pltpu.SemaphoreType.DMA,
        pltpu.SemaphoreType.DMA,
        pltpu.SemaphoreType.DMA,
        pltpu.SemaphoreType.DMA,
    ],
)
def _sc_user_gather(user_ids_hbm, user_blocks_hbm, user_out_hbm,
                    uids_v, ubuf0, ubuf1, ugs0, ugs1, uws0, uws1):
    wid = lax.axis_index("s") * _NC + lax.axis_index("c")
    base = wid * _BPW
    pltpu.sync_copy(user_ids_hbm.at[pl.ds(base, _BPW)], uids_v)

    ubufs = (ubuf0, ubuf1)
    ugs = (ugs0, ugs1)
    uws = (uws0, uws1)

    def _user_group(g, buf, gsem):
        for t in range(_UG // 16):
            vec = uids_v[pl.ds((g * (_UG // 16) + t) * 16, 16)]
            blkv = lax.shift_right_logical(vec, 3)
            subv = lax.bitwise_and(vec, 7)
            for r in range(16):
                blk = jnp.squeeze(lax.slice(blkv, (r,), (r + 1,)))
                sub = jnp.squeeze(lax.slice(subv, (r,), (r + 1,)))
                pltpu.async_copy(
                    user_blocks_hbm.at[blk, sub], buf.at[t * 16 + r], gsem)
        def _drain(r, _):
            pltpu.make_async_copy(
                user_blocks_hbm.at[0, 0], buf.at[r], gsem).wait()
            return 0

        lax.fori_loop(0, _UG, _drain, 0)

    uwh = [None, None]
    for g in range(_NUG):
        b = g % 2
        if uwh[b] is not None:
            uwh[b].wait()
        _user_group(g, ubufs[b], ugs[b])
        uwh[b] = pltpu.async_copy(
            ubufs[b], user_out_hbm.at[pl.ds(base + g * _UG, _UG)], uws[b])
    uwh[0].wait()
    uwh[1].wait()


_BM = 2048


def _mm_body(x_ref, w_ref, o_ref):
    o_ref[...] = jnp.dot(x_ref[...], w_ref[...],
                         preferred_element_type=jnp.float32)


def _tc_project(item_rows, item_proj):
    return pl.pallas_call(
        _mm_body,
        grid=(B // _BM,),
        in_specs=[
            pl.BlockSpec((_BM, FEAT), lambda i: (i, 0)),
            pl.BlockSpec((FEAT, EMBED), lambda i: (0, 0)),
        ],
        out_specs=pl.BlockSpec((_BM, EMBED), lambda i: (i, 0)),
        out_shape=jax.ShapeDtypeStruct((B, EMBED), jnp.float32),
    )(item_rows, item_proj)


def kernel(user_ids, item_ids, user_table, item_feats, item_proj):
    item_rows = _sc_item_gather(item_ids.astype(jnp.int32), item_feats)
    item_emb = _tc_project(item_rows, item_proj)
    user_table, item_rows = lax.optimization_barrier((user_table, item_rows))
    user_blocks = jnp.reshape(user_table, (NUM_USERS // 8, 8, EMBED))
    user_emb = _sc_user_gather(user_ids.astype(jnp.int32), user_blocks)
    return (user_emb, item_emb)


# speedup vs baseline: 1.4333x; 1.4333x over previous
import functools

import jax
import jax.numpy as jnp
from jax import lax
from jax.experimental import pallas as pl
from jax.experimental.pallas import tpu as pltpu
from jax.experimental.pallas import tpu_sc as plsc

B = 16384
EMBED = 64
FEAT = 256
NUM_USERS = 1000000

_info = plsc.get_sparse_core_info()
_NC = _info.num_cores
_NS = _info.num_subcores
_NW = _NC * _NS
_BPW = B // _NW

_ICH = 64
_NIC = _BPW // _ICH
_UG = 64
_NUG = _BPW // _UG

_mesh = plsc.VectorSubcoreMesh(core_axis_name="c", subcore_axis_name="s")


@functools.partial(
    pl.kernel,
    mesh=_mesh,
    compiler_params=pltpu.CompilerParams(needs_layout_passes=False,
                                         skip_device_barrier=True),
    out_type=jax.ShapeDtypeStruct((B, FEAT), jnp.float32),
    scratch_types=[
        pltpu.VMEM((_BPW,), jnp.int32),
        pltpu.VMEM((_ICH, FEAT), jnp.float32),
        pltpu.VMEM((_ICH, FEAT), jnp.float32),
        pltpu.SemaphoreType.DMA,
        pltpu.SemaphoreType.DMA,
        pltpu.SemaphoreType.DMA,
        pltpu.SemaphoreType.DMA,
    ],
)
def _sc_item_gather(item_ids_hbm, item_feats_hbm, item_rows_hbm,
                    iids_v, ibuf0, ibuf1, igs0, igs1, iws0, iws1):
    wid = lax.axis_index("s") * _NC + lax.axis_index("c")
    base = wid * _BPW
    pltpu.sync_copy(item_ids_hbm.at[pl.ds(base, _BPW)], iids_v)

    ibufs = (ibuf0, ibuf1)
    igs = (igs0, igs1)
    iws = (iws0, iws1)

    def _gather(g, buf, sem):
        return pltpu.async_copy(
            item_feats_hbm.at[iids_v.at[pl.ds(g * _ICH, _ICH)]], buf, sem)

    igh = [None, None]
    iwh = [None, None]
    igh[0] = _gather(0, ibufs[0], igs[0])
    for g in range(_NIC):
        b = g % 2
        if g + 1 < _NIC:
            if iwh[1 - b] is not None:
                iwh[1 - b].wait()
            igh[1 - b] = _gather(g + 1, ibufs[1 - b], igs[1 - b])
        igh[b].wait()
        iwh[b] = pltpu.async_copy(
            ibufs[b], item_rows_hbm.at[pl.ds(base + g * _ICH, _ICH)], iws[b])
    iwh[0].wait()
    iwh[1].wait()


@functools.partial(
    pl.kernel,
    mesh=_mesh,
    compiler_params=pltpu.CompilerParams(needs_layout_passes=False,
                                         skip_device_barrier=True),
    out_type=jax.ShapeDtypeStruct((B, EMBED), jnp.float32),
    scratch_types=[
        pltpu.VMEM((_BPW,), jnp.int32),
        pltpu.VMEM((_UG, EMBED), jnp.float32),
        pltpu.VMEM((_UG, EMBED), jnp.float32),
        pltpu.SemaphoreType.DMA,
        pltpu.SemaphoreType.DMA,
        pltpu.SemaphoreType.DMA,
        pltpu.SemaphoreType.DMA,
    ],
)
def _sc_user_gather(user_ids_hbm, user_blocks_hbm, user_out_hbm,
                    uids_v, ubuf0, ubuf1, ugs0, ugs1, uws0, uws1):
    wid = lax.axis_index("s") * _NC + lax.axis_index("c")
    base = wid * _BPW
    pltpu.sync_copy(user_ids_hbm.at[pl.ds(base, _BPW)], uids_v)

    ubufs = (ubuf0, ubuf1)
    ugs = (ugs0, ugs1)
    uws = (uws0, uws1)

    def _user_group(g, buf, gsem):
        for t in range(_UG // 16):
            vec = uids_v[pl.ds((g * (_UG // 16) + t) * 16, 16)]
            blkv = lax.shift_right_logical(vec, 3)
            subv = lax.bitwise_and(vec, 7)
            for r in range(16):
                blk = jnp.squeeze(lax.slice(blkv, (r,), (r + 1,)))
                sub = jnp.squeeze(lax.slice(subv, (r,), (r + 1,)))
                pltpu.async_copy(
                    user_blocks_hbm.at[blk, sub], buf.at[t * 16 + r], gsem)
        def _drain(r, _):
            pltpu.make_async_copy(
                user_blocks_hbm.at[0, 0], buf.at[r], gsem).wait()
            return 0

        lax.fori_loop(0, _UG, _drain, 0)

    uwh = [None, None]
    for g in range(_NUG):
        b = g % 2
        if uwh[b] is not None:
            uwh[b].wait()
        _user_group(g, ubufs[b], ugs[b])
        uwh[b] = pltpu.async_copy(
            ubufs[b], user_out_hbm.at[pl.ds(base + g * _UG, _UG)], uws[b])
    uwh[0].wait()
    uwh[1].wait()


_BM = 2048


def _mm_body(x_ref, w_ref, o_ref):
    o_ref[...] = jnp.dot(x_ref[...], w_ref[...],
                         preferred_element_type=jnp.float32)


def _tc_project(item_rows, item_proj):
    return pl.pallas_call(
        _mm_body,
        grid=(B // _BM,),
        in_specs=[
            pl.BlockSpec((_BM, FEAT), lambda i: (i, 0)),
            pl.BlockSpec((FEAT, EMBED), lambda i: (0, 0)),
        ],
        out_specs=pl.BlockSpec((_BM, EMBED), lambda i: (i, 0)),
        out_shape=jax.ShapeDtypeStruct((B, EMBED), jnp.float32),
    )(item_rows, item_proj)


def kernel(user_ids, item_ids, user_table, item_feats, item_proj):
    item_rows = _sc_item_gather(item_ids.astype(jnp.int32), item_feats)
    item_emb = _tc_project(item_rows, item_proj)
    user_blocks = jnp.reshape(user_table, (NUM_USERS // 8, 8, EMBED))
    user_emb = _sc_user_gather(user_ids.astype(jnp.int32), user_blocks)
    return (user_emb, item_emb)


# speedup vs baseline: 1.4477x; 1.0100x over previous
import functools

import jax
import jax.numpy as jnp
from jax import lax
from jax.experimental import pallas as pl
from jax.experimental.pallas import tpu as pltpu
from jax.experimental.pallas import tpu_sc as plsc

B = 16384
EMBED = 64
FEAT = 256
NUM_USERS = 1000000

_info = plsc.get_sparse_core_info()
_NC = _info.num_cores
_NS = _info.num_subcores
_NW = _NC * _NS
_BPW = B // _NW

_ICH = 64
_NIC = _BPW // _ICH
_UG = 64
_NUG = _BPW // _UG

_mesh = plsc.VectorSubcoreMesh(core_axis_name="c", subcore_axis_name="s")


@functools.partial(
    pl.kernel,
    mesh=_mesh,
    compiler_params=pltpu.CompilerParams(needs_layout_passes=False,
                                         skip_device_barrier=True),
    out_type=jax.ShapeDtypeStruct((B, FEAT), jnp.float32),
    scratch_types=[
        pltpu.VMEM((_BPW,), jnp.int32),
        pltpu.VMEM((_ICH, FEAT), jnp.float32),
        pltpu.VMEM((_ICH, FEAT), jnp.float32),
        pltpu.SemaphoreType.DMA,
        pltpu.SemaphoreType.DMA,
        pltpu.SemaphoreType.DMA,
        pltpu.SemaphoreType.DMA,
    ],
)
def _sc_item_gather(item_ids_hbm, item_feats_hbm, item_rows_hbm,
                    iids_v, ibuf0, ibuf1, igs0, igs1, iws0, iws1):
    wid = lax.axis_index("s") * _NC + lax.axis_index("c")
    base = wid * _BPW
    pltpu.sync_copy(item_ids_hbm.at[pl.ds(base, _BPW)], iids_v)

    ibufs = (ibuf0, ibuf1)
    igs = (igs0, igs1)
    iws = (iws0, iws1)

    def _gather(g, buf, sem):
        return pltpu.async_copy(
            item_feats_hbm.at[iids_v.at[pl.ds(g * _ICH, _ICH)]], buf, sem)

    igh = [None, None]
    iwh = [None, None]
    igh[0] = _gather(0, ibufs[0], igs[0])
    for g in range(_NIC):
        b = g % 2
        if g + 1 < _NIC:
            if iwh[1 - b] is not None:
                iwh[1 - b].wait()
            igh[1 - b] = _gather(g + 1, ibufs[1 - b], igs[1 - b])
        igh[b].wait()
        iwh[b] = pltpu.async_copy(
            ibufs[b], item_rows_hbm.at[pl.ds(base + g * _ICH, _ICH)], iws[b])
    iwh[0].wait()
    iwh[1].wait()


@functools.partial(
    pl.kernel,
    mesh=_mesh,
    compiler_params=pltpu.CompilerParams(needs_layout_passes=False,
                                         skip_device_barrier=True),
    out_type=jax.ShapeDtypeStruct((B, EMBED), jnp.float32),
    scratch_types=[
        pltpu.VMEM((_BPW,), jnp.int32),
        pltpu.VMEM((_UG, EMBED), jnp.float32),
        pltpu.VMEM((_UG, EMBED), jnp.float32),
        pltpu.VMEM((_UG, EMBED), jnp.float32),
        pltpu.SemaphoreType.DMA,
        pltpu.SemaphoreType.DMA,
        pltpu.SemaphoreType.DMA,
        pltpu.SemaphoreType.DMA,
        pltpu.SemaphoreType.DMA,
        pltpu.SemaphoreType.DMA,
    ],
)
def _sc_user_gather(user_ids_hbm, user_blocks_hbm, user_out_hbm,
                    uids_v, ubuf0, ubuf1, ubuf2,
                    ugs0, ugs1, ugs2, uws0, uws1, uws2):
    wid = lax.axis_index("s") * _NC + lax.axis_index("c")
    base = wid * _BPW
    pltpu.sync_copy(user_ids_hbm.at[pl.ds(base, _BPW)], uids_v)

    ubufs = (ubuf0, ubuf1, ubuf2)
    ugs = (ugs0, ugs1, ugs2)
    uws = (uws0, uws1, uws2)

    def _fire(g, buf, gsem):
        for t in range(_UG // 16):
            vec = uids_v[pl.ds((g * (_UG // 16) + t) * 16, 16)]
            blkv = lax.shift_right_logical(vec, 3)
            subv = lax.bitwise_and(vec, 7)
            for r in range(16):
                blk = jnp.squeeze(lax.slice(blkv, (r,), (r + 1,)))
                sub = jnp.squeeze(lax.slice(subv, (r,), (r + 1,)))
                pltpu.async_copy(
                    user_blocks_hbm.at[blk, sub], buf.at[t * 16 + r], gsem)

    def _drain_all(buf, gsem):
        def _drain(r, _):
            pltpu.make_async_copy(
                user_blocks_hbm.at[0, 0], buf.at[r], gsem).wait()
            return 0

        lax.fori_loop(0, _UG, _drain, 0)

    uwh = [None, None, None]
    _fire(0, ubufs[0], ugs[0])
    for g in range(_NUG):
        b = g % 3
        if g + 1 < _NUG:
            nb = (g + 1) % 3
            if uwh[nb] is not None:
                uwh[nb].wait()
            _fire(g + 1, ubufs[nb], ugs[nb])
        _drain_all(ubufs[b], ugs[b])
        uwh[b] = pltpu.async_copy(
            ubufs[b], user_out_hbm.at[pl.ds(base + g * _UG, _UG)], uws[b])
    for h in uwh:
        if h is not None:
            h.wait()


_BM = 2048


def _mm_body(x_ref, w_ref, o_ref):
    o_ref[...] = jnp.dot(x_ref[...], w_ref[...],
                         preferred_element_type=jnp.float32)


def _tc_project(item_rows, item_proj):
    return pl.pallas_call(
        _mm_body,
        grid=(B // _BM,),
        in_specs=[
            pl.BlockSpec((_BM, FEAT), lambda i: (i, 0)),
            pl.BlockSpec((FEAT, EMBED), lambda i: (0, 0)),
        ],
        out_specs=pl.BlockSpec((_BM, EMBED), lambda i: (i, 0)),
        out_shape=jax.ShapeDtypeStruct((B, EMBED), jnp.float32),
    )(item_rows, item_proj)


def kernel(user_ids, item_ids, user_table, item_feats, item_proj):
    item_rows = _sc_item_gather(item_ids.astype(jnp.int32), item_feats)
    item_emb = _tc_project(item_rows, item_proj)
    user_blocks = jnp.reshape(user_table, (NUM_USERS // 8, 8, EMBED))
    user_emb = _sc_user_gather(user_ids.astype(jnp.int32), user_blocks)
    return (user_emb, item_emb)


# speedup vs baseline: 1.4530x; 1.0037x over previous
import functools

import jax
import jax.numpy as jnp
from jax import lax
from jax.experimental import pallas as pl
from jax.experimental.pallas import tpu as pltpu
from jax.experimental.pallas import tpu_sc as plsc

B = 16384
EMBED = 64
FEAT = 256
NUM_USERS = 1000000

_info = plsc.get_sparse_core_info()
_NC = _info.num_cores
_NS = _info.num_subcores
_NW = _NC * _NS
_BPW = B // _NW

_ICH = 128
_NIC = _BPW // _ICH
_UG = 64
_NUG = _BPW // _UG

_mesh = plsc.VectorSubcoreMesh(core_axis_name="c", subcore_axis_name="s")


@functools.partial(
    pl.kernel,
    mesh=_mesh,
    compiler_params=pltpu.CompilerParams(needs_layout_passes=False,
                                         skip_device_barrier=True),
    out_type=jax.ShapeDtypeStruct((B, FEAT), jnp.float32),
    scratch_types=[
        pltpu.VMEM((_BPW,), jnp.int32),
        pltpu.VMEM((_ICH, FEAT), jnp.float32),
        pltpu.VMEM((_ICH, FEAT), jnp.float32),
        pltpu.SemaphoreType.DMA,
        pltpu.SemaphoreType.DMA,
        pltpu.SemaphoreType.DMA,
        pltpu.SemaphoreType.DMA,
    ],
)
def _sc_item_gather(item_ids_hbm, item_feats_hbm, item_rows_hbm,
                    iids_v, ibuf0, ibuf1, igs0, igs1, iws0, iws1):
    wid = lax.axis_index("s") * _NC + lax.axis_index("c")
    base = wid * _BPW
    pltpu.sync_copy(item_ids_hbm.at[pl.ds(base, _BPW)], iids_v)

    ibufs = (ibuf0, ibuf1)
    igs = (igs0, igs1)
    iws = (iws0, iws1)

    def _gather(g, buf, sem):
        return pltpu.async_copy(
            item_feats_hbm.at[iids_v.at[pl.ds(g * _ICH, _ICH)]], buf, sem)

    igh = [None, None]
    iwh = [None, None]
    igh[0] = _gather(0, ibufs[0], igs[0])
    for g in range(_NIC):
        b = g % 2
        if g + 1 < _NIC:
            if iwh[1 - b] is not None:
                iwh[1 - b].wait()
            igh[1 - b] = _gather(g + 1, ibufs[1 - b], igs[1 - b])
        igh[b].wait()
        iwh[b] = pltpu.async_copy(
            ibufs[b], item_rows_hbm.at[pl.ds(base + g * _ICH, _ICH)], iws[b])
    iwh[0].wait()
    iwh[1].wait()


@functools.partial(
    pl.kernel,
    mesh=_mesh,
    compiler_params=pltpu.CompilerParams(needs_layout_passes=False,
                                         skip_device_barrier=True),
    out_type=jax.ShapeDtypeStruct((B, EMBED), jnp.float32),
    scratch_types=[
        pltpu.VMEM((_BPW,), jnp.int32),
        pltpu.VMEM((_UG, EMBED), jnp.float32),
        pltpu.VMEM((_UG, EMBED), jnp.float32),
        pltpu.VMEM((_UG, EMBED), jnp.float32),
        pltpu.SemaphoreType.DMA,
        pltpu.SemaphoreType.DMA,
        pltpu.SemaphoreType.DMA,
        pltpu.SemaphoreType.DMA,
        pltpu.SemaphoreType.DMA,
        pltpu.SemaphoreType.DMA,
    ],
)
def _sc_user_gather(user_ids_hbm, user_blocks_hbm, user_out_hbm,
                    uids_v, ubuf0, ubuf1, ubuf2,
                    ugs0, ugs1, ugs2, uws0, uws1, uws2):
    wid = lax.axis_index("s") * _NC + lax.axis_index("c")
    base = wid * _BPW
    pltpu.sync_copy(user_ids_hbm.at[pl.ds(base, _BPW)], uids_v)

    ubufs = (ubuf0, ubuf1, ubuf2)
    ugs = (ugs0, ugs1, ugs2)
    uws = (uws0, uws1, uws2)

    def _fire(g, buf, gsem):
        for t in range(_UG // 16):
            vec = uids_v[pl.ds((g * (_UG // 16) + t) * 16, 16)]
            blkv = lax.shift_right_logical(vec, 3)
            subv = lax.bitwise_and(vec, 7)
            for r in range(16):
                blk = jnp.squeeze(lax.slice(blkv, (r,), (r + 1,)))
                sub = jnp.squeeze(lax.slice(subv, (r,), (r + 1,)))
                pltpu.async_copy(
                    user_blocks_hbm.at[blk, sub], buf.at[t * 16 + r], gsem)

    def _drain_all(buf, gsem):
        def _drain(r, _):
            pltpu.make_async_copy(
                user_blocks_hbm.at[0, 0], buf.at[r], gsem).wait()
            return 0

        lax.fori_loop(0, _UG, _drain, 0)

    uwh = [None, None, None]
    _fire(0, ubufs[0], ugs[0])
    for g in range(_NUG):
        b = g % 3
        if g + 1 < _NUG:
            nb = (g + 1) % 3
            if uwh[nb] is not None:
                uwh[nb].wait()
            _fire(g + 1, ubufs[nb], ugs[nb])
        _drain_all(ubufs[b], ugs[b])
        uwh[b] = pltpu.async_copy(
            ubufs[b], user_out_hbm.at[pl.ds(base + g * _UG, _UG)], uws[b])
    for h in uwh:
        if h is not None:
            h.wait()


_BM = 2048


def _mm_body(x_ref, w_ref, o_ref):
    o_ref[...] = jnp.dot(x_ref[...], w_ref[...],
                         preferred_element_type=jnp.float32)


def _tc_project(item_rows, item_proj):
    return pl.pallas_call(
        _mm_body,
        grid=(B // _BM,),
        in_specs=[
            pl.BlockSpec((_BM, FEAT), lambda i: (i, 0)),
            pl.BlockSpec((FEAT, EMBED), lambda i: (0, 0)),
        ],
        out_specs=pl.BlockSpec((_BM, EMBED), lambda i: (i, 0)),
        out_shape=jax.ShapeDtypeStruct((B, EMBED), jnp.float32),
    )(item_rows, item_proj)


def kernel(user_ids, item_ids, user_table, item_feats, item_proj):
    item_rows = _sc_item_gather(item_ids.astype(jnp.int32), item_feats)
    item_emb = _tc_project(item_rows, item_proj)
    user_blocks = jnp.reshape(user_table, (NUM_USERS // 8, 8, EMBED))
    user_emb = _sc_user_gather(user_ids.astype(jnp.int32), user_blocks)
    return (user_emb, item_emb)
